# Initial kernel scaffold; baseline (speedup 1.0000x reference)
import jax
import jax.numpy as jnp
from jax.experimental import pallas as pl


def kernel(x, word_ebd, pos_ebd):
    raise NotImplementedError("write your pallas kernel here")


# speedup vs baseline: 5.6311x; 5.6311x over previous
import functools

import jax
import jax.numpy as jnp
from jax import lax
from jax.experimental import pallas as pl
from jax.experimental.pallas import tpu as pltpu
from jax.experimental.pallas import tpu_sc as plsc

B, T, D, V = 16384, 12, 24, 29
NTOK = B * T
NW = 32
TOK_W = NTOK // NW
CHUNK = 128
SUPER = 1536
NSUP = TOK_W // SUPER
NG = SUPER // CHUNK


def _ebd_body(x_hbm, fused_hbm, out_hbm, idx_v, outbuf, sem):
    cid = lax.axis_index("c")
    sid = lax.axis_index("s")
    wid = sid * 2 + cid
    base = wid * TOK_W

    pltpu.sync_copy(x_hbm.at[pl.ds(base, TOK_W)], idx_v)

    lane = lax.iota(jnp.int32, 16)
    offs = [((lane + 16 * r) % 12) * 29 for r in range(3)]

    def add_body(i, carry):
        j0 = i * 48
        for r in range(3):
            sl = pl.ds(j0 + 16 * r, 16)
            idx_v[sl] = idx_v[sl] + offs[r]
        return carry

    lax.fori_loop(0, TOK_W // 48, add_body, 0)

    def super_body(s, carry):
        t0 = s * SUPER
        handles = []
        for k in range(NG):
            h = pltpu.async_copy(
                fused_hbm.at[idx_v.at[pl.ds(t0 + k * CHUNK, CHUNK)]],
                outbuf.at[pl.ds(k * CHUNK, CHUNK)],
                sem,
            )
            handles.append(h)
        for h in handles:
            h.wait()
        pltpu.sync_copy(outbuf, out_hbm.at[pl.ds(base + t0, SUPER)])
        return carry

    lax.fori_loop(0, NSUP, super_body, 0)


@jax.jit
def _ebd_gather(xf, fused):
    mesh = plsc.VectorSubcoreMesh(core_axis_name="c", subcore_axis_name="s")
    run = functools.partial(
        pl.kernel,
        out_type=jax.ShapeDtypeStruct((NTOK, D), jnp.float32),
        mesh=mesh,
        scratch_types=[
            pltpu.VMEM((TOK_W,), jnp.int32),
            pltpu.VMEM((SUPER, D), jnp.float32),
            pltpu.SemaphoreType.DMA,
        ],
        compiler_params=pltpu.CompilerParams(use_tc_tiling_on_sc=False),
    )(_ebd_body)
    return run(xf, fused)


def kernel(x, word_ebd, pos_ebd):
    fused = (pos_ebd[:, None, :] + word_ebd[None, :, :]).reshape(T * V, D)
    xf = x.reshape(NTOK).astype(jnp.int32)
    out = _ebd_gather(xf, fused)
    return out.reshape(B, T, D)


# speedup vs baseline: 5.8530x; 1.0394x over previous
import functools

import jax
import jax.numpy as jnp
from jax import lax
from jax.experimental import pallas as pl
from jax.experimental.pallas import tpu as pltpu
from jax.experimental.pallas import tpu_sc as plsc

B, T, D, V = 16384, 12, 24, 29
NTOK = B * T
NW = 32
ROWS_W = B // NW
TOK_W = NTOK // NW
CHUNK = 128
RSUP = 128
SUPER = RSUP * T
NSUP = ROWS_W // RSUP
NG = SUPER // CHUNK


def _ebd_body(x_hbm, fused_hbm, out_hbm, idxT, outbuf, sem):
    cid = lax.axis_index("c")
    sid = lax.axis_index("s")
    wid = sid * 2 + cid
    rbase = wid * ROWS_W

    pltpu.sync_copy(x_hbm.at[:, pl.ds(rbase, ROWS_W)], idxT)

    def q_body(q, carry):
        for t in range(1, T):
            sl = pl.ds(q * 16, 16)
            idxT[t, sl] = idxT[t, sl] + (29 * t)
        return carry

    lax.fori_loop(0, ROWS_W // 16, q_body, 0)

    def super_body(s, carry):
        r0 = s * RSUP
        handles = []
        for t in range(T):
            h = pltpu.async_copy(
                fused_hbm.at[idxT.at[t, pl.ds(r0, RSUP)]],
                outbuf.at[t],
                sem,
            )
            handles.append(h)
        for h in handles:
            h.wait()
        for t in range(T):
            pltpu.sync_copy(outbuf.at[t],
                            out_hbm.at[pl.ds(rbase + r0, RSUP), t])
        return carry

    lax.fori_loop(0, NSUP, super_body, 0)


@jax.jit
def _ebd_gather(xi, fused):
    mesh = plsc.VectorSubcoreMesh(core_axis_name="c", subcore_axis_name="s")
    run = functools.partial(
        pl.kernel,
        out_type=jax.ShapeDtypeStruct((B, T, D), jnp.float32),
        mesh=mesh,
        scratch_types=[
            pltpu.VMEM((T, ROWS_W), jnp.int32),
            pltpu.VMEM((T, RSUP, D), jnp.float32),
            pltpu.SemaphoreType.DMA,
        ],
        compiler_params=pltpu.CompilerParams(use_tc_tiling_on_sc=False),
    )(_ebd_body)
    return run(xi, fused)


def kernel(x, word_ebd, pos_ebd):
    fused = (pos_ebd[:, None, :] + word_ebd[None, :, :]).reshape(T * V, D)
    return _ebd_gather(x.T.astype(jnp.int32), fused)


# speedup vs baseline: 17.7562x; 3.0337x over previous
import functools

import jax
import jax.numpy as jnp
from jax import lax
from jax.experimental import pallas as pl
from jax.experimental.pallas import tpu as pltpu
from jax.experimental.pallas import tpu_sc as plsc

B, T, D, V = 16384, 12, 24, 29
NW = 32
ROWS_W = B // NW
DT = D // 8
BT_W = ROWS_W // 128
NSL = ROWS_W // 16


def _ebd_body(x_hbm, lut_hbm, out_hbm, xv, lut_v, buf0, buf1, sem, fsem):
    cid = lax.axis_index("c")
    sid = lax.axis_index("s")
    wid = sid * 2 + cid
    rbase = wid * ROWS_W

    pltpu.sync_copy(x_hbm.at[:, pl.ds(rbase, ROWS_W)], xv)
    pltpu.sync_copy(lut_hbm, lut_v)

    bufs = [buf0, buf1]

    def compute_t(t, buf):
        def sl_body(i, carry):
            xvec = xv[t, pl.ds(i * 16, 16)]
            bt = i // 8
            lo = (i % 8) * 16
            for d in range(D):
                vals = plsc.load_gather(lut_v, [xvec + ((t * D + d) * V)])
                buf[(d // 8), bt, (d % 8), pl.ds(lo, 16)] = vals
            return carry

        lax.fori_loop(0, NSL, sl_body, 0)

    def flush_t(t, buf):
        return [
            pltpu.async_copy(
                buf.at[dt],
                out_hbm.at[t, dt, pl.ds(wid * BT_W, BT_W)],
                fsem,
            )
            for dt in range(DT)
        ]

    pending = []
    for t in range(T):
        buf = bufs[t % 2]
        if t >= 2:
            for h in pending[t - 2]:
                h.wait()
        compute_t(t, buf)
        pending.append(flush_t(t, buf))
    for hs in pending[T - 2:]:
        for h in hs:
            h.wait()


@jax.jit
def _ebd_gather(xi, lut):
    mesh = plsc.VectorSubcoreMesh(core_axis_name="c", subcore_axis_name="s")
    run = functools.partial(
        pl.kernel,
        out_type=jax.ShapeDtypeStruct((T, DT, B // 128, 8, 128), jnp.float32),
        mesh=mesh,
        scratch_types=[
            pltpu.VMEM((T, ROWS_W), jnp.int32),
            pltpu.VMEM((T * D * V,), jnp.float32),
            pltpu.VMEM((DT, BT_W, 8, 128), jnp.float32),
            pltpu.VMEM((DT, BT_W, 8, 128), jnp.float32),
            pltpu.SemaphoreType.DMA,
            pltpu.SemaphoreType.DMA,
        ],
        compiler_params=pltpu.CompilerParams(
            use_tc_tiling_on_sc=False, needs_layout_passes=False),
    )(_ebd_body)
    return run(xi, lut)


def kernel(x, word_ebd, pos_ebd):
    lut = (pos_ebd[:, None, :] + word_ebd[None, :, :]).transpose(0, 2, 1)
    out5 = _ebd_gather(x.T.astype(jnp.int32), lut.reshape(T * D * V))
    return out5.transpose(2, 4, 0, 1, 3).reshape(B, T, D)


# speedup vs baseline: 30.3585x; 1.7097x over previous
import functools

import jax
import jax.numpy as jnp
from jax import lax
from jax.experimental import pallas as pl
from jax.experimental.pallas import tpu as pltpu
from jax.experimental.pallas import tpu_sc as plsc

B, T, D, V = 16384, 12, 24, 29
NW = 32
ROWS_W = B // NW
DT = D // 8
BT_W = ROWS_W // 128
NSL = ROWS_W // 16


def _ebd_body(x_hbm, lut_hbm, out_hbm, xv, lut_v, buf0, buf1, sem, fsem):
    cid = lax.axis_index("c")
    sid = lax.axis_index("s")
    wid = sid * 2 + cid
    rbase = wid * ROWS_W

    pltpu.sync_copy(x_hbm.at[:, pl.ds(rbase, ROWS_W)], xv)
    pltpu.sync_copy(lut_hbm, lut_v)

    bufs = [buf0, buf1]

    def compute_t(t, buf):
        @plsc.parallel_loop(0, NSL, unroll=2)
        def sl_body(i):
            xvec = xv[t, pl.ds(i * 16, 16)]
            bt = i // 8
            lo = (i % 8) * 16
            for d in range(D):
                vals = plsc.load_gather(lut_v, [xvec + ((t * D + d) * V)])
                buf[(d // 8), bt, (d % 8), pl.ds(lo, 16)] = vals

    def flush_t(t, buf):
        return [
            pltpu.async_copy(
                buf.at[dt],
                out_hbm.at[t, dt, pl.ds(wid * BT_W, BT_W)],
                fsem,
            )
            for dt in range(DT)
        ]

    pending = []
    for t in range(T):
        buf = bufs[t % 2]
        if t >= 2:
            for h in pending[t - 2]:
                h.wait()
        compute_t(t, buf)
        pending.append(flush_t(t, buf))
    for hs in pending[T - 2:]:
        for h in hs:
            h.wait()


@jax.jit
def _ebd_gather(xi, lut):
    mesh = plsc.VectorSubcoreMesh(core_axis_name="c", subcore_axis_name="s")
    run = functools.partial(
        pl.kernel,
        out_type=jax.ShapeDtypeStruct((T, DT, B // 128, 8, 128), jnp.float32),
        mesh=mesh,
        scratch_types=[
            pltpu.VMEM((T, ROWS_W), jnp.int32),
            pltpu.VMEM((T * D * V,), jnp.float32),
            pltpu.VMEM((DT, BT_W, 8, 128), jnp.float32),
            pltpu.VMEM((DT, BT_W, 8, 128), jnp.float32),
            pltpu.SemaphoreType.DMA,
            pltpu.SemaphoreType.DMA,
        ],
        compiler_params=pltpu.CompilerParams(
            use_tc_tiling_on_sc=False, needs_layout_passes=False),
    )(_ebd_body)
    return run(xi, lut)


def kernel(x, word_ebd, pos_ebd):
    lut = (pos_ebd[:, None, :] + word_ebd[None, :, :]).transpose(0, 2, 1)
    out5 = _ebd_gather(x.T.astype(jnp.int32), lut.reshape(T * D * V))
    return out5.transpose(2, 4, 0, 1, 3).reshape(B, T, D)
